# baseline (speedup 1.0000x reference)
import functools

import jax
import jax.numpy as jnp
from jax import lax
from jax.experimental import pallas as pl
from jax.experimental.pallas import tpu as pltpu
from jax.experimental.pallas import tpu_sc as plsc

_CHUNK = 128


@functools.lru_cache(maxsize=None)
def _make_gather(V, D, B):
    info = plsc.get_sparse_core_info()
    NC, NS = info.num_cores, info.num_subcores
    NW = NC * NS
    assert B % NW == 0
    b_per_w = B // NW
    n_chunks = b_per_w // _CHUNK
    assert n_chunks * _CHUNK == b_per_w
    mesh = plsc.VectorSubcoreMesh(core_axis_name="c", subcore_axis_name="s")

    @functools.partial(
        pl.kernel,
        mesh=mesh,
        compiler_params=pltpu.CompilerParams(use_tc_tiling_on_sc=False),
        out_type=jax.ShapeDtypeStruct((B, D), jnp.float32),
        scratch_types=[
            pltpu.VMEM((b_per_w,), jnp.int32),
            pltpu.VMEM((b_per_w, D), jnp.float32),
            pltpu.SemaphoreType.DMA,
        ],
    )
    def gather_kernel(idx_hbm, table_hbm, out_hbm, idx_v, rows_v, sem):
        wid = lax.axis_index("s") * NC + lax.axis_index("c")
        base = wid * b_per_w
        pltpu.sync_copy(idx_hbm.at[pl.ds(base, b_per_w)], idx_v)
        copies = [
            pltpu.async_copy(
                table_hbm.at[idx_v.at[pl.ds(c * _CHUNK, _CHUNK)]],
                rows_v.at[pl.ds(c * _CHUNK, _CHUNK)],
                sem,
            )
            for c in range(n_chunks)
        ]
        for cp in copies:
            cp.wait()
        pltpu.sync_copy(rows_v, out_hbm.at[pl.ds(base, b_per_w)])

    return gather_kernel


def kernel(style_label, embedding):
    (B,) = style_label.shape
    V, D = embedding.shape
    return _make_gather(V, D, B)(style_label.astype(jnp.int32), embedding)


# speedup vs baseline: 1.7323x; 1.7323x over previous
import functools

import jax
import jax.numpy as jnp
from jax import lax
from jax.experimental import pallas as pl
from jax.experimental.pallas import tpu as pltpu
from jax.experimental.pallas import tpu_sc as plsc

_L = 16


@functools.lru_cache(maxsize=None)
def _make_gather(V, D, B):
    info = plsc.get_sparse_core_info()
    NC, NS = info.num_cores, info.num_subcores
    NW = NC * NS
    assert B % (NW * _L) == 0
    b_per_w = B // NW
    n_grp = b_per_w // _L
    mesh = plsc.VectorSubcoreMesh(core_axis_name="c", subcore_axis_name="s")

    @functools.partial(
        pl.kernel,
        mesh=mesh,
        out_type=jax.ShapeDtypeStruct((B, D), jnp.float32),
        scratch_types=[
            pltpu.VMEM((b_per_w,), jnp.int32),
            pltpu.VMEM((b_per_w, D), jnp.float32),
            pltpu.SemaphoreType.DMA,
        ],
    )
    def gather_kernel(idx_hbm, table_hbm, out_hbm, idx_v, rows_v, sem):
        wid = lax.axis_index("s") * NC + lax.axis_index("c")
        base = wid * b_per_w
        pltpu.sync_copy(idx_hbm.at[pl.ds(base, b_per_w)], idx_v)

        def fire_group(g, carry):
            idx16 = idx_v[pl.ds(_L * g, _L)]
            for j in range(_L):
                r = idx16[j]
                pltpu.async_copy(
                    table_hbm.at[pl.ds(r, 1)],
                    rows_v.at[pl.ds(_L * g + j, 1)],
                    sem,
                )
            return carry

        lax.fori_loop(0, n_grp, fire_group, jnp.int32(0))
        pltpu.make_async_copy(
            table_hbm.at[pl.ds(0, b_per_w)], rows_v, sem).wait()
        pltpu.sync_copy(rows_v, out_hbm.at[pl.ds(base, b_per_w)])

    return gather_kernel


def kernel(style_label, embedding):
    (B,) = style_label.shape
    V, D = embedding.shape
    return _make_gather(V, D, B)(style_label.astype(jnp.int32), embedding)


# speedup vs baseline: 2.9123x; 1.6812x over previous
import functools

import jax
import jax.numpy as jnp
from jax import lax
from jax.experimental import pallas as pl
from jax.experimental.pallas import tpu as pltpu
from jax.experimental.pallas import tpu_sc as plsc

_L = 16
_CW = 512
_CSH = 9
_NB = 64


@functools.lru_cache(maxsize=None)
def _make_gather(V, D, B):
    info = plsc.get_sparse_core_info()
    NC, NS = info.num_cores, info.num_subcores
    NW = NC * NS
    assert D % _L == 0 and B % _L == 0 and V > _CW
    n_chunks = (V + _CW - 1) // _CW
    assert n_chunks <= (_NB - 1) * NW
    CAP = B + _L
    mesh = plsc.VectorSubcoreMesh(core_axis_name="c", subcore_axis_name="s")

    @functools.partial(
        pl.kernel,
        mesh=mesh,
        compiler_params=pltpu.CompilerParams(needs_layout_passes=False),
        out_type=jax.ShapeDtypeStruct((B + 64, 128), jnp.float32),
        scratch_types=[
            pltpu.VMEM((B,), jnp.int32),
            pltpu.VMEM((CAP,), jnp.int32),
            pltpu.VMEM((CAP,), jnp.int32),
            pltpu.VMEM((_NB,), jnp.int32),
            pltpu.VMEM((_NB,), jnp.int32),
            pltpu.VMEM((_NB,), jnp.int32),
            pltpu.VMEM((D, _CW), jnp.float32),
            pltpu.VMEM((D, _CW), jnp.float32),
            pltpu.VMEM((_L, 128), jnp.float32),
            pltpu.VMEM((_L,), jnp.int32),
            pltpu.SemaphoreType.DMA,
            pltpu.SemaphoreType.DMA,
            pltpu.SemaphoreType.DMA,
        ],
    )
    def gather_kernel(idx_hbm, tT_hbm, staged_hbm,
                      idx_all, srt_r, srt_b,
                      hist, starts, cur, slab_a, slab_b,
                      hitbuf, bidx, sem_a, sem_b, ssem):
        w = lax.axis_index("s") * NC + lax.axis_index("c")
        lanes = lax.iota(jnp.int32, _L)
        zeros16 = jnp.zeros((_L,), jnp.int32)
        pltpu.sync_copy(idx_hbm, idx_all)

        for q in range(_NB // _L):
            hist[pl.ds(_L * q, _L)] = zeros16

        def histb(g, carry):
            r16 = idx_all[pl.ds(_L * g, _L)]
            cid = jnp.right_shift(r16, _CSH)
            mine = (cid & (NW - 1)) == w
            gc = jnp.where(mine, jnp.right_shift(r16, _CSH + 5), _NB - 1)
            plsc.addupdate_scatter(hist, [gc],
                                   jnp.where(mine, 1, 0).astype(jnp.int32))
            return carry

        lax.fori_loop(0, B // _L, histb, jnp.int32(0))

        carry16 = zeros16
        for q in range(_NB // _L):
            h16 = hist[pl.ds(_L * q, _L)]
            inc = plsc.cumsum(h16)
            starts[pl.ds(_L * q, _L)] = carry16 + inc - h16
            carry16 = carry16 + jnp.full((_L,), 0, jnp.int32) + inc[_L - 1]
        for q in range(_NB // _L):
            cur[pl.ds(_L * q, _L)] = starts[pl.ds(_L * q, _L)]

        def place(g, carry):
            r16 = idx_all[pl.ds(_L * g, _L)]
            cid = jnp.right_shift(r16, _CSH)
            mine = (cid & (NW - 1)) == w
            gc16 = jnp.where(mine, jnp.right_shift(r16, _CSH + 5), _NB - 1)
            rank, last = plsc.scan_count(gc16, mine)
            base16 = plsc.load_gather(cur, [gc16])
            dest = base16 + rank - 1
            plsc.store_scatter(srt_r, [dest], r16, mask=mine)
            plsc.store_scatter(srt_b, [dest], lanes + _L * g, mask=mine)
            plsc.store_scatter(cur, [gc16], dest + 1,
                               mask=jnp.logical_and(last, mine))
            return carry

        lax.fori_loop(0, B // _L, place, jnp.int32(0))

        n_full = V // _CW
        tail_w = V - n_full * _CW
        n_t = (n_full - w + NW - 1) // NW
        feat = [lanes + _L * q for q in range(D // _L)]

        def fire(t, slab, sem):
            coff = pl.multiple_of((w + NW * t) * _CW, _CW)
            pltpu.async_copy(tT_hbm.at[:, pl.ds(coff, _CW)], slab, sem)

        def drain(slab, sem):
            pltpu.make_async_copy(
                tT_hbm.at[:, pl.ds(0, _CW)], slab, sem).wait()

        def process(t, slab, coff):
            t16 = zeros16 + t
            s_t = plsc.load_gather(hist, [t16])[0]
            st_t = plsc.load_gather(starts, [t16])[0]
            n_g = (s_t + _L - 1) // _L

            def grp(g, carry):
                gbase = st_t + _L * g
                r16 = srt_r[pl.ds(gbase, _L)]
                b16 = srt_b[pl.ds(gbase, _L)]
                valid = (lanes + _L * g) < s_t
                col = jnp.where(valid, r16 - coff, 0)
                bs = jnp.where(valid, b16, B + w)
                bidx[...] = bs
                for e in range(_L):
                    ce = col[e]
                    for q in range(D // _L):
                        vals = plsc.load_gather(slab, [feat[q], zeros16 + ce])
                        hitbuf[e, pl.ds(_L * q, _L)] = vals
                pltpu.async_copy(hitbuf, staged_hbm.at[bidx], ssem).wait()
                return carry

            lax.fori_loop(0, n_g, grp, jnp.int32(0))

        fire(jnp.int32(0), slab_a, sem_a)

        def scan_body(u, carry):
            ta = 2 * u
            tb = 2 * u + 1

            @pl.when(tb < n_t)
            def _():
                fire(tb, slab_b, sem_b)

            drain(slab_a, sem_a)
            process(ta, slab_a, pl.multiple_of((w + NW * ta) * _CW, _CW))

            @pl.when(ta + 2 < n_t)
            def _():
                fire(ta + 2, slab_a, sem_a)

            @pl.when(tb < n_t)
            def _():
                drain(slab_b, sem_b)
                process(tb, slab_b, pl.multiple_of((w + NW * tb) * _CW, _CW))

            return carry

        lax.fori_loop(0, (n_t + 1) // 2, scan_body, jnp.int32(0))


    return gather_kernel


def kernel(style_label, embedding):
    (B,) = style_label.shape
    V, D = embedding.shape
    idx = style_label.astype(jnp.int32)
    staged = _make_gather(V, D, B)(idx, embedding.T)
    out = staged[:B, :D]
    n_full = V // _CW
    tail = V - n_full * _CW
    if tail:
        tail_tab = embedding[n_full * _CW:]
        is_tail = idx >= (n_full * _CW)
        tail_rows = jnp.take(
            tail_tab, jnp.where(is_tail, idx - n_full * _CW, 0), axis=0)
        out = jnp.where(is_tail[:, None], tail_rows, out)
    return out


# speedup vs baseline: 2.9262x; 1.0048x over previous
import functools

import jax
import jax.numpy as jnp
from jax import lax
from jax.experimental import pallas as pl
from jax.experimental.pallas import tpu as pltpu
from jax.experimental.pallas import tpu_sc as plsc

_L = 16
_CW = 512
_CSH = 9
_NB = 64


@functools.lru_cache(maxsize=None)
def _make_gather(V, D, B):
    info = plsc.get_sparse_core_info()
    NC, NS = info.num_cores, info.num_subcores
    NW = NC * NS
    assert D % _L == 0 and B % _L == 0 and V > _CW
    n_chunks = (V + _CW - 1) // _CW
    assert n_chunks <= (_NB - 1) * NW
    CAP = B + _L
    mesh = plsc.VectorSubcoreMesh(core_axis_name="c", subcore_axis_name="s")

    @functools.partial(
        pl.kernel,
        mesh=mesh,
        compiler_params=pltpu.CompilerParams(needs_layout_passes=False),
        out_type=jax.ShapeDtypeStruct((B + 64, 128), jnp.float32),
        scratch_types=[
            pltpu.VMEM((B,), jnp.int32),
            pltpu.VMEM((CAP,), jnp.int32),
            pltpu.VMEM((CAP,), jnp.int32),
            pltpu.VMEM((_NB,), jnp.int32),
            pltpu.VMEM((_NB,), jnp.int32),
            pltpu.VMEM((_NB,), jnp.int32),
            pltpu.VMEM((D, _CW), jnp.float32),
            pltpu.VMEM((D, _CW), jnp.float32),
            pltpu.VMEM((_L, 128), jnp.float32),
            pltpu.VMEM((_L, 128), jnp.float32),
            pltpu.VMEM((_L,), jnp.int32),
            pltpu.VMEM((_L,), jnp.int32),
            pltpu.SemaphoreType.DMA,
            pltpu.SemaphoreType.DMA,
            pltpu.SemaphoreType.DMA,
            pltpu.SemaphoreType.DMA,
        ],
    )
    def gather_kernel(idx_hbm, tT_hbm, staged_hbm,
                      idx_all, srt_r, srt_b,
                      hist, starts, cur, slab_a, slab_b,
                      hitbuf_a, hitbuf_b, bidx_a, bidx_b,
                      sem_a, sem_b, ssem_a, ssem_b):
        w = lax.axis_index("s") * NC + lax.axis_index("c")
        lanes = lax.iota(jnp.int32, _L)
        zeros16 = jnp.zeros((_L,), jnp.int32)
        pltpu.sync_copy(idx_hbm, idx_all)

        for q in range(_NB // _L):
            hist[pl.ds(_L * q, _L)] = zeros16

        def histb(g, carry):
            r16 = idx_all[pl.ds(_L * g, _L)]
            cid = jnp.right_shift(r16, _CSH)
            mine = (cid & (NW - 1)) == w
            gc = jnp.where(mine, jnp.right_shift(r16, _CSH + 5), _NB - 1)
            plsc.addupdate_scatter(hist, [gc],
                                   jnp.where(mine, 1, 0).astype(jnp.int32))
            return carry

        lax.fori_loop(0, B // _L, histb, jnp.int32(0))

        carry16 = zeros16
        for q in range(_NB // _L):
            h16 = hist[pl.ds(_L * q, _L)]
            inc = plsc.cumsum(h16)
            starts[pl.ds(_L * q, _L)] = carry16 + inc - h16
            carry16 = carry16 + jnp.full((_L,), 0, jnp.int32) + inc[_L - 1]
        for q in range(_NB // _L):
            cur[pl.ds(_L * q, _L)] = starts[pl.ds(_L * q, _L)]

        def place(g, carry):
            r16 = idx_all[pl.ds(_L * g, _L)]
            cid = jnp.right_shift(r16, _CSH)
            mine = (cid & (NW - 1)) == w
            gc16 = jnp.where(mine, jnp.right_shift(r16, _CSH + 5), _NB - 1)
            rank, last = plsc.scan_count(gc16, mine)
            base16 = plsc.load_gather(cur, [gc16])
            dest = base16 + rank - 1
            plsc.store_scatter(srt_r, [dest], r16, mask=mine)
            plsc.store_scatter(srt_b, [dest], lanes + _L * g, mask=mine)
            plsc.store_scatter(cur, [gc16], dest + 1,
                               mask=jnp.logical_and(last, mine))
            return carry

        lax.fori_loop(0, B // _L, place, jnp.int32(0))

        n_full = V // _CW
        tail_w = V - n_full * _CW
        n_t = (n_full - w + NW - 1) // NW
        feat = [lanes + _L * q for q in range(D // _L)]

        def fire(t, slab, sem):
            coff = pl.multiple_of((w + NW * t) * _CW, _CW)
            pltpu.async_copy(tT_hbm.at[:, pl.ds(coff, _CW)], slab, sem)

        def drain(slab, sem):
            pltpu.make_async_copy(
                tT_hbm.at[:, pl.ds(0, _CW)], slab, sem).wait()

        def process(t, slab, coff, hitbuf, bidx, ssem):
            t16 = zeros16 + t
            s_t = plsc.load_gather(hist, [t16])[0]
            st_t = plsc.load_gather(starts, [t16])[0]
            n_g = (s_t + _L - 1) // _L

            def grp(g, carry):
                pltpu.make_async_copy(hitbuf, staged_hbm.at[bidx], ssem).wait()
                gbase = st_t + _L * g
                r16 = srt_r[pl.ds(gbase, _L)]
                b16 = srt_b[pl.ds(gbase, _L)]
                valid = (lanes + _L * g) < s_t
                col = jnp.where(valid, r16 - coff, 0)
                bs = jnp.where(valid, b16, B + w)
                bidx[...] = bs
                for e in range(_L):
                    ce = col[e]
                    for q in range(D // _L):
                        vals = plsc.load_gather(slab, [feat[q], zeros16 + ce])
                        hitbuf[e, pl.ds(_L * q, _L)] = vals
                pltpu.async_copy(hitbuf, staged_hbm.at[bidx], ssem)
                return carry

            lax.fori_loop(0, n_g, grp, jnp.int32(0))

        bidx_a[...] = zeros16 + (B + w)
        bidx_b[...] = zeros16 + (B + w)
        pltpu.async_copy(hitbuf_a, staged_hbm.at[bidx_a], ssem_a)
        pltpu.async_copy(hitbuf_b, staged_hbm.at[bidx_b], ssem_b)

        fire(jnp.int32(0), slab_a, sem_a)

        def scan_body(u, carry):
            ta = 2 * u
            tb = 2 * u + 1

            @pl.when(tb < n_t)
            def _():
                fire(tb, slab_b, sem_b)

            drain(slab_a, sem_a)
            process(ta, slab_a, pl.multiple_of((w + NW * ta) * _CW, _CW),
                    hitbuf_a, bidx_a, ssem_a)

            @pl.when(ta + 2 < n_t)
            def _():
                fire(ta + 2, slab_a, sem_a)

            @pl.when(tb < n_t)
            def _():
                drain(slab_b, sem_b)
                process(tb, slab_b, pl.multiple_of((w + NW * tb) * _CW, _CW),
                        hitbuf_b, bidx_b, ssem_b)

            return carry

        lax.fori_loop(0, (n_t + 1) // 2, scan_body, jnp.int32(0))

        pltpu.make_async_copy(hitbuf_a, staged_hbm.at[bidx_a], ssem_a).wait()
        pltpu.make_async_copy(hitbuf_b, staged_hbm.at[bidx_b], ssem_b).wait()


    return gather_kernel


def kernel(style_label, embedding):
    (B,) = style_label.shape
    V, D = embedding.shape
    idx = style_label.astype(jnp.int32)
    staged = _make_gather(V, D, B)(idx, embedding.T)
    out = staged[:B, :D]
    n_full = V // _CW
    tail = V - n_full * _CW
    if tail:
        tail_tab = embedding[n_full * _CW:]
        is_tail = idx >= (n_full * _CW)
        tail_rows = jnp.take(
            tail_tab, jnp.where(is_tail, idx - n_full * _CW, 0), axis=0)
        out = jnp.where(is_tail[:, None], tail_rows, out)
    return out
